# baseline (speedup 1.0000x reference)
import functools

import jax
import jax.numpy as jnp
from jax import lax
from jax.experimental import pallas as pl
from jax.experimental.pallas import tpu as pltpu
from jax.experimental.pallas import tpu_sc as plsc

N = 10000
E = 320000
D = 128
H = 128
O = 64
B = 64

NC = 2
NS = 16
NW = NC * NS

CHUNK = 128
G = 16
NG = 5
CHUNKS = G * NG
EPAD = NW * CHUNKS * CHUNK

DEGW = 128
NROWS = 10240
RPS = NROWS // NS
RBLK = 2048
NBLK = NROWS // RBLK

_mesh = plsc.VectorSubcoreMesh(core_axis_name="c", subcore_axis_name="s",
                               num_cores=NC, num_subcores=NS)


def _sc_deg_body(dst_hbm, ones_hbm, z16_hbm, out_hbm, didx, vones, accum):
    ci = lax.axis_index("c")
    si = lax.axis_index("s")
    w = ci * NS + si
    sl = pl.ds(si * RPS, RPS)
    pltpu.sync_copy(z16_hbm.at[sl], accum.at[sl])
    pltpu.sync_copy(ones_hbm, vones)
    pltpu.sync_copy(dst_hbm.at[w], didx)
    plsc.subcore_barrier()

    @pl.loop(0, CHUNKS)
    def _(j):
        pltpu.sync_copy(vones, accum.at[didx.at[j]], add=True)

    plsc.subcore_barrier()
    pltpu.sync_copy(accum.at[sl], out_hbm.at[ci].at[sl])


def _sc_edge_body(hs_hbm, src_hbm, dst_hbm, out_hbm, sidx, didx,
                  gbuf, accum, sem):
    ci = lax.axis_index("c")
    si = lax.axis_index("s")
    w = ci * NS + si
    sl = pl.ds(si * RPS, RPS)
    pltpu.sync_copy(hs_hbm.at[sl], accum.at[sl])
    pltpu.sync_copy(src_hbm.at[w], sidx)
    pltpu.sync_copy(dst_hbm.at[w], didx)
    plsc.subcore_barrier()

    @pl.loop(0, CHUNKS)
    def _(j):
        pltpu.async_copy(hs_hbm.at[sidx.at[j]], gbuf, sem).wait()
        pltpu.sync_copy(gbuf, accum.at[didx.at[j]], add=True)

    plsc.subcore_barrier()
    pltpu.sync_copy(accum.at[sl], out_hbm.at[ci].at[sl])


def _make_sc_deg(interpret=False):
    return functools.partial(
        pl.kernel,
        out_type=jax.ShapeDtypeStruct((NC, NROWS, DEGW), jnp.float32),
        mesh=_mesh,
        scratch_types=[
            pltpu.VMEM((CHUNKS, CHUNK), jnp.int32),
            pltpu.VMEM((CHUNK, DEGW), jnp.float32),
            pltpu.VMEM_SHARED((NROWS, DEGW), jnp.float32),
        ],
        interpret=interpret,
    )(_sc_deg_body)


def _make_sc_edge(interpret=False):
    return functools.partial(
        pl.kernel,
        out_type=jax.ShapeDtypeStruct((NC, NROWS, 128), jnp.float32),
        mesh=_mesh,
        scratch_types=[
            pltpu.VMEM((CHUNKS, CHUNK), jnp.int32),
            pltpu.VMEM((CHUNKS, CHUNK), jnp.int32),
            pltpu.VMEM((CHUNK, 128), jnp.float32),
            pltpu.VMEM_SHARED((NROWS, 128), jnp.float32),
            pltpu.SemaphoreType.DMA,
        ],
        interpret=interpret,
    )(_sc_edge_body)


_sc_deg = _make_sc_deg()
_sc_edge = _make_sc_edge()


def _mm(a, b):
    return jax.lax.dot_general(
        a, b, (((1,), (0,)), ((), ())),
        precision=jax.lax.Precision.HIGHEST,
        preferred_element_type=jnp.float32,
    )


def _dinv_of(deg2_ref):
    deg = deg2_ref[0, :, 0:1] + deg2_ref[1, :, 0:1] + 1.0
    return jax.lax.rsqrt(deg)


def _tc_h1_body(x_ref, w_ref, o_ref):
    o_ref[...] = _mm(x_ref[...], w_ref[...])


def _tc_h1(x, W1):
    return pl.pallas_call(
        _tc_h1_body,
        grid=(NBLK,),
        in_specs=[
            pl.BlockSpec((RBLK, D), lambda i: (i, 0)),
            pl.BlockSpec((D, H), lambda i: (0, 0)),
        ],
        out_specs=pl.BlockSpec((RBLK, H), lambda i: (i, 0)),
        out_shape=jax.ShapeDtypeStruct((NROWS, H), jnp.float32),
    )(x, W1)


def _tc_scale_body(h_ref, deg2_ref, o_ref):
    o_ref[...] = h_ref[...] * _dinv_of(deg2_ref)


def _tc_scale(h1, deg2):
    return pl.pallas_call(
        _tc_scale_body,
        grid=(NBLK,),
        in_specs=[
            pl.BlockSpec((RBLK, H), lambda i: (i, 0)),
            pl.BlockSpec((NC, RBLK, DEGW), lambda i: (0, i, 0)),
        ],
        out_specs=pl.BlockSpec((RBLK, H), lambda i: (i, 0)),
        out_shape=jax.ShapeDtypeStruct((NROWS, H), jnp.float32),
    )(h1, deg2)


def _tc_layer_body(e_ref, hs_ref, deg2_ref, w_ref, b_ref, o_ref):
    dinv = _dinv_of(deg2_ref)
    agg = e_ref[0] + e_ref[1] - hs_ref[...]
    z = agg * dinv + b_ref[...]
    r = jnp.maximum(z, 0.0)
    o_ref[...] = _mm(r, w_ref[...]) * dinv


def _tc_layer(e1, hs1, deg2, W2, b1):
    return pl.pallas_call(
        _tc_layer_body,
        grid=(NBLK,),
        in_specs=[
            pl.BlockSpec((NC, RBLK, H), lambda i: (0, i, 0)),
            pl.BlockSpec((RBLK, H), lambda i: (i, 0)),
            pl.BlockSpec((NC, RBLK, DEGW), lambda i: (0, i, 0)),
            pl.BlockSpec((H, H), lambda i: (0, 0)),
            pl.BlockSpec((1, H), lambda i: (0, 0)),
        ],
        out_specs=pl.BlockSpec((RBLK, H), lambda i: (i, 0)),
        out_shape=jax.ShapeDtypeStruct((NROWS, H), jnp.float32),
    )(e1, hs1, deg2, W2, b1)


def _tc_final_body(e_ref, hs_ref, deg2_ref, bat_ref, b_ref, fcw_ref, fcb_ref,
                   o_ref, sums, cnt):
    i = pl.program_id(0)
    dinv = _dinv_of(deg2_ref)
    agg = e_ref[0] + e_ref[1] - hs_ref[...]
    z = agg * dinv + b_ref[...]
    r = jnp.maximum(z, 0.0)

    bids = bat_ref[0, 0, :]
    seg = lax.broadcasted_iota(jnp.int32, (B, RBLK), 0)
    mask = (bids[None, :] == seg).astype(jnp.float32)

    @pl.when(i == 0)
    def _():
        sums[...] = jnp.zeros_like(sums)
        cnt[...] = jnp.zeros_like(cnt)

    sums[...] += _mm(mask, r)
    cnt[...] += jnp.broadcast_to(
        jnp.sum(mask, axis=1, keepdims=True), cnt.shape)

    @pl.when(i == NBLK - 1)
    def _():
        pooled = sums[...] / jnp.maximum(cnt[...], 1.0)
        o_ref[...] = _mm(pooled, fcw_ref[...]) + fcb_ref[...]


def _tc_final(e2, hs2, deg2, batch3, b2, fcW, fcb):
    return pl.pallas_call(
        _tc_final_body,
        grid=(NBLK,),
        in_specs=[
            pl.BlockSpec((NC, RBLK, H), lambda i: (0, i, 0)),
            pl.BlockSpec((RBLK, H), lambda i: (i, 0)),
            pl.BlockSpec((NC, RBLK, DEGW), lambda i: (0, i, 0)),
            pl.BlockSpec((1, 1, RBLK), lambda i: (i, 0, 0)),
            pl.BlockSpec((1, H), lambda i: (0, 0)),
            pl.BlockSpec((H, O), lambda i: (0, 0)),
            pl.BlockSpec((1, O), lambda i: (0, 0)),
        ],
        out_specs=pl.BlockSpec((B, O), lambda i: (0, 0)),
        out_shape=jax.ShapeDtypeStruct((B, O), jnp.float32),
        scratch_shapes=[
            pltpu.VMEM((B, H), jnp.float32),
            pltpu.VMEM((B, H), jnp.float32),
        ],
    )(e2, hs2, deg2, batch3, b2, fcW, fcb)


@jax.jit
def kernel(x, edge_index, batch, W1, b1, W2, b2, fcW, fcb):
    src = edge_index[0]
    dst = edge_index[1]
    pad_dst = N + jnp.arange(EPAD - E, dtype=jnp.int32) % (NROWS - N)
    srcp = jnp.concatenate(
        [src, jnp.zeros((EPAD - E,), jnp.int32)]).reshape(NW, CHUNKS, CHUNK)
    dstp = jnp.concatenate([dst, pad_dst]).reshape(NW, CHUNKS, CHUNK)
    dstp3 = dstp

    xp = jnp.concatenate([x, jnp.zeros((NROWS - N, D), jnp.float32)], axis=0)
    batp = jnp.concatenate(
        [batch, jnp.full((NROWS - N,), B, jnp.int32)]).reshape(NBLK, 1, RBLK)

    ones128 = jnp.ones((CHUNK, DEGW), jnp.float32)
    z16 = jnp.zeros((NROWS, DEGW), jnp.float32)

    deg2 = _sc_deg(dstp3, ones128, z16)
    h1 = _tc_h1(xp, W1)
    hs1 = _tc_scale(h1, deg2)
    e1 = _sc_edge(hs1, srcp, dstp)
    hs2 = _tc_layer(e1, hs1, deg2, W2, b1.reshape(1, H))
    e2 = _sc_edge(hs2, srcp, dstp)
    return _tc_final(e2, hs2, deg2, batp, b2.reshape(1, H),
                     fcW, fcb.reshape(1, O))


# speedup vs baseline: 2.7287x; 2.7287x over previous
import functools

import jax
import jax.numpy as jnp
from jax import lax
from jax.experimental import pallas as pl
from jax.experimental.pallas import tpu as pltpu
from jax.experimental.pallas import tpu_sc as plsc

N = 10000
E = 320000
D = 128
H = 128
O = 64
B = 64

NC = 2
NS = 16
NW = NC * NS

CHUNK = 128
G = 16
NG = 5
CHUNKS = G * NG
EPAD = NW * CHUNKS * CHUNK

DEGW = 128
NROWS = 10240
RPS = NROWS // NS
RBLK = 2048
NBLK = NROWS // RBLK

_mesh = plsc.VectorSubcoreMesh(core_axis_name="c", subcore_axis_name="s",
                               num_cores=NC, num_subcores=NS)


def _sc_deg_body(dst_hbm, ones_hbm, z16_hbm, out_hbm, didx, vones, accum):
    ci = lax.axis_index("c")
    si = lax.axis_index("s")
    w = ci * NS + si
    sl = pl.ds(si * RPS, RPS)
    pltpu.sync_copy(z16_hbm.at[sl], accum.at[sl])
    pltpu.sync_copy(ones_hbm, vones)
    pltpu.sync_copy(dst_hbm.at[w], didx)
    plsc.subcore_barrier()

    @pl.loop(0, CHUNKS)
    def _(j):
        pltpu.sync_copy(vones, accum.at[didx.at[j]], add=True)

    plsc.subcore_barrier()
    pltpu.sync_copy(accum.at[sl], out_hbm.at[ci].at[sl])


def _sc_edge_body(hs_hbm, src_hbm, dst_hbm, out_hbm, sidx, didx,
                  gbuf, accum, sem):
    ci = lax.axis_index("c")
    si = lax.axis_index("s")
    w = ci * NS + si
    sl = pl.ds(si * RPS, RPS)
    pltpu.sync_copy(hs_hbm.at[sl], accum.at[sl])
    pltpu.sync_copy(src_hbm.at[w], sidx)
    pltpu.sync_copy(dst_hbm.at[w], didx)
    plsc.subcore_barrier()

    @pl.loop(0, CHUNKS)
    def _(j):
        pltpu.async_copy(hs_hbm.at[sidx.at[j]], gbuf, sem).wait()
        pltpu.sync_copy(gbuf, accum.at[didx.at[j]], add=True)

    plsc.subcore_barrier()
    pltpu.sync_copy(accum.at[sl], out_hbm.at[ci].at[sl])


def _make_sc_deg(interpret=False):
    return functools.partial(
        pl.kernel,
        out_type=jax.ShapeDtypeStruct((NC, NROWS, DEGW), jnp.float32),
        mesh=_mesh,
        scratch_types=[
            pltpu.VMEM((CHUNKS, CHUNK), jnp.int32),
            pltpu.VMEM((CHUNK, DEGW), jnp.float32),
            pltpu.VMEM_SHARED((NROWS, DEGW), jnp.float32),
        ],
        interpret=interpret,
    )(_sc_deg_body)


def _make_sc_edge(interpret=False):
    return functools.partial(
        pl.kernel,
        out_type=jax.ShapeDtypeStruct((NC, NROWS, 128), jnp.float32),
        mesh=_mesh,
        scratch_types=[
            pltpu.VMEM((CHUNKS, CHUNK), jnp.int32),
            pltpu.VMEM((CHUNKS, CHUNK), jnp.int32),
            pltpu.VMEM((CHUNK, 128), jnp.float32),
            pltpu.VMEM_SHARED((NROWS, 128), jnp.float32),
            pltpu.SemaphoreType.DMA,
        ],
        interpret=interpret,
    )(_sc_edge_body)


_sc_deg = _make_sc_deg()
_sc_edge = _make_sc_edge()


def _mm(a, b):
    return jax.lax.dot_general(
        a, b, (((1,), (0,)), ((), ())),
        precision=jax.lax.Precision.HIGHEST,
        preferred_element_type=jnp.float32,
    )


def _dinv_of(deg2_ref):
    deg = deg2_ref[0, :, 0:1] + deg2_ref[1, :, 0:1] + 1.0
    return jax.lax.rsqrt(deg)


def _tc_h1_body(x_ref, w_ref, o_ref):
    o_ref[...] = _mm(x_ref[...], w_ref[...])


def _tc_h1(x, W1):
    return pl.pallas_call(
        _tc_h1_body,
        grid=(NBLK,),
        in_specs=[
            pl.BlockSpec((RBLK, D), lambda i: (i, 0)),
            pl.BlockSpec((D, H), lambda i: (0, 0)),
        ],
        out_specs=pl.BlockSpec((RBLK, H), lambda i: (i, 0)),
        out_shape=jax.ShapeDtypeStruct((NROWS, H), jnp.float32),
    )(x, W1)


def _tc_scale_body(h_ref, deg2_ref, o_ref):
    o_ref[...] = h_ref[...] * _dinv_of(deg2_ref)


def _tc_scale(h1, deg2):
    return pl.pallas_call(
        _tc_scale_body,
        grid=(NBLK,),
        in_specs=[
            pl.BlockSpec((RBLK, H), lambda i: (i, 0)),
            pl.BlockSpec((NC, RBLK, DEGW), lambda i: (0, i, 0)),
        ],
        out_specs=pl.BlockSpec((RBLK, H), lambda i: (i, 0)),
        out_shape=jax.ShapeDtypeStruct((NROWS, H), jnp.float32),
    )(h1, deg2)


def _tc_layer_body(e_ref, hs_ref, deg2_ref, w_ref, b_ref, o_ref):
    dinv = _dinv_of(deg2_ref)
    agg = e_ref[0] + e_ref[1] - hs_ref[...]
    z = agg * dinv + b_ref[...]
    r = jnp.maximum(z, 0.0)
    o_ref[...] = _mm(r, w_ref[...]) * dinv


def _tc_layer(e1, hs1, deg2, W2, b1):
    return pl.pallas_call(
        _tc_layer_body,
        grid=(NBLK,),
        in_specs=[
            pl.BlockSpec((NC, RBLK, H), lambda i: (0, i, 0)),
            pl.BlockSpec((RBLK, H), lambda i: (i, 0)),
            pl.BlockSpec((NC, RBLK, DEGW), lambda i: (0, i, 0)),
            pl.BlockSpec((H, H), lambda i: (0, 0)),
            pl.BlockSpec((1, H), lambda i: (0, 0)),
        ],
        out_specs=pl.BlockSpec((RBLK, H), lambda i: (i, 0)),
        out_shape=jax.ShapeDtypeStruct((NROWS, H), jnp.float32),
    )(e1, hs1, deg2, W2, b1)


def _tc_final_body(e_ref, hs_ref, deg2_ref, bat_ref, b_ref, fcw_ref, fcb_ref,
                   o_ref, sums, cnt):
    i = pl.program_id(0)
    dinv = _dinv_of(deg2_ref)
    agg = e_ref[0] + e_ref[1] - hs_ref[...]
    z = agg * dinv + b_ref[...]
    r = jnp.maximum(z, 0.0)

    bids = bat_ref[0, 0, :]
    seg = lax.broadcasted_iota(jnp.int32, (B, RBLK), 0)
    mask = (bids[None, :] == seg).astype(jnp.float32)

    @pl.when(i == 0)
    def _():
        sums[...] = jnp.zeros_like(sums)
        cnt[...] = jnp.zeros_like(cnt)

    sums[...] += _mm(mask, r)
    cnt[...] += jnp.broadcast_to(
        jnp.sum(mask, axis=1, keepdims=True), cnt.shape)

    @pl.when(i == NBLK - 1)
    def _():
        pooled = sums[...] / jnp.maximum(cnt[...], 1.0)
        o_ref[...] = _mm(pooled, fcw_ref[...]) + fcb_ref[...]


def _tc_final(e2, hs2, deg2, batch3, b2, fcW, fcb):
    return pl.pallas_call(
        _tc_final_body,
        grid=(NBLK,),
        in_specs=[
            pl.BlockSpec((NC, RBLK, H), lambda i: (0, i, 0)),
            pl.BlockSpec((RBLK, H), lambda i: (i, 0)),
            pl.BlockSpec((NC, RBLK, DEGW), lambda i: (0, i, 0)),
            pl.BlockSpec((1, 1, RBLK), lambda i: (i, 0, 0)),
            pl.BlockSpec((1, H), lambda i: (0, 0)),
            pl.BlockSpec((H, O), lambda i: (0, 0)),
            pl.BlockSpec((1, O), lambda i: (0, 0)),
        ],
        out_specs=pl.BlockSpec((B, O), lambda i: (0, 0)),
        out_shape=jax.ShapeDtypeStruct((B, O), jnp.float32),
        scratch_shapes=[
            pltpu.VMEM((B, H), jnp.float32),
            pltpu.VMEM((B, H), jnp.float32),
        ],
    )(e2, hs2, deg2, batch3, b2, fcW, fcb)


@jax.jit
def kernel(x, edge_index, batch, W1, b1, W2, b2, fcW, fcb):
    src = edge_index[0]
    dst = edge_index[1]
    pad_idx = N + jnp.arange(EPAD - E, dtype=jnp.int32) % (NROWS - N)
    srcp = jnp.concatenate([src, pad_idx]).reshape(NW, CHUNKS, CHUNK)
    dstp = jnp.concatenate([dst, pad_idx]).reshape(NW, CHUNKS, CHUNK)
    dstp3 = dstp

    xp = jnp.concatenate([x, jnp.zeros((NROWS - N, D), jnp.float32)], axis=0)
    batp = jnp.concatenate(
        [batch, jnp.full((NROWS - N,), B, jnp.int32)]).reshape(NBLK, 1, RBLK)

    ones128 = jnp.ones((CHUNK, DEGW), jnp.float32)
    z16 = jnp.zeros((NROWS, DEGW), jnp.float32)

    deg2 = _sc_deg(dstp3, ones128, z16)
    h1 = _tc_h1(xp, W1)
    hs1 = _tc_scale(h1, deg2)
    e1 = _sc_edge(hs1, srcp, dstp)
    hs2 = _tc_layer(e1, hs1, deg2, W2, b1.reshape(1, H))
    e2 = _sc_edge(hs2, srcp, dstp)
    return _tc_final(e2, hs2, deg2, batp, b2.reshape(1, H),
                     fcW, fcb.reshape(1, O))


# speedup vs baseline: 3.3102x; 1.2131x over previous
import functools

import jax
import jax.numpy as jnp
from jax import lax
from jax.experimental import pallas as pl
from jax.experimental.pallas import tpu as pltpu
from jax.experimental.pallas import tpu_sc as plsc

N = 10000
E = 320000
D = 128
H = 128
O = 64
B = 64

NC = 2
NS = 16
NW = NC * NS

CHUNK = 128
G = 16
NG = 5
CHUNKS = G * NG
EPAD = NW * CHUNKS * CHUNK

DEGW = 128
NROWS = 10240
RPS = NROWS // NS
RBLK = 2048
NBLK = NROWS // RBLK

_mesh = plsc.VectorSubcoreMesh(core_axis_name="c", subcore_axis_name="s",
                               num_cores=NC, num_subcores=NS)


def _sc_deg_body(dst_hbm, ones_hbm, z16_hbm, out_hbm, didx, vones, accum):
    ci = lax.axis_index("c")
    si = lax.axis_index("s")
    w = ci * NS + si
    sl = pl.ds(si * RPS, RPS)
    pltpu.sync_copy(z16_hbm.at[sl], accum.at[sl])
    pltpu.sync_copy(ones_hbm, vones)
    pltpu.sync_copy(dst_hbm.at[w], didx)
    plsc.subcore_barrier()

    @pl.loop(0, CHUNKS)
    def _(j):
        pltpu.sync_copy(vones, accum.at[didx.at[j]], add=True)

    plsc.subcore_barrier()
    pltpu.sync_copy(accum.at[sl], out_hbm.at[ci].at[sl])


def _sc_edge_body(hs_hbm, src_hbm, dst_hbm, out_hbm, sidx, didx,
                  gbuf0, gbuf1, accum, sem0, sem1, semsi, semdi):
    ci = lax.axis_index("c")
    si = lax.axis_index("s")
    w = ci * NS + si
    sl = pl.ds(si * RPS, RPS)
    pltpu.sync_copy(hs_hbm.at[sl], accum.at[sl])
    pltpu.async_copy(src_hbm.at[w].at[0], sidx.at[0], semsi)
    pltpu.async_copy(dst_hbm.at[w].at[0], didx.at[0], semdi)
    plsc.subcore_barrier()

    for g in range(NG):
        b = g & 1
        sg = sidx.at[b]
        dg = didx.at[b]
        pltpu.make_async_copy(src_hbm.at[w].at[g], sg, semsi).wait()
        pltpu.make_async_copy(dst_hbm.at[w].at[g], dg, semdi).wait()
        if g + 1 < NG:
            pltpu.async_copy(src_hbm.at[w].at[g + 1], sidx.at[1 - b], semsi)
            pltpu.async_copy(dst_hbm.at[w].at[g + 1], didx.at[1 - b], semdi)
        pltpu.async_copy(hs_hbm.at[sg.at[0]], gbuf0, sem0)

        @pl.loop(0, G // 2)
        def _(p):
            j = 2 * p
            pltpu.make_async_copy(hs_hbm.at[sg.at[j]], gbuf0, sem0).wait()
            pltpu.async_copy(hs_hbm.at[sg.at[j + 1]], gbuf1, sem1)
            pltpu.sync_copy(gbuf0, accum.at[dg.at[j]], add=True)
            pltpu.make_async_copy(hs_hbm.at[sg.at[j + 1]], gbuf1, sem1).wait()

            @pl.when(p < G // 2 - 1)
            def _():
                pltpu.async_copy(hs_hbm.at[sg.at[j + 2]], gbuf0, sem0)

            pltpu.sync_copy(gbuf1, accum.at[dg.at[j + 1]], add=True)

    plsc.subcore_barrier()
    pltpu.sync_copy(accum.at[sl], out_hbm.at[ci].at[sl])


def _make_sc_deg(interpret=False):
    return functools.partial(
        pl.kernel,
        out_type=jax.ShapeDtypeStruct((NC, NROWS, DEGW), jnp.float32),
        mesh=_mesh,
        scratch_types=[
            pltpu.VMEM((CHUNKS, CHUNK), jnp.int32),
            pltpu.VMEM((CHUNK, DEGW), jnp.float32),
            pltpu.VMEM_SHARED((NROWS, DEGW), jnp.float32),
        ],
        interpret=interpret,
    )(_sc_deg_body)


def _make_sc_edge(interpret=False):
    return functools.partial(
        pl.kernel,
        out_type=jax.ShapeDtypeStruct((NC, NROWS, 128), jnp.float32),
        mesh=_mesh,
        scratch_types=[
            pltpu.VMEM((2, G, CHUNK), jnp.int32),
            pltpu.VMEM((2, G, CHUNK), jnp.int32),
            pltpu.VMEM((CHUNK, 128), jnp.float32),
            pltpu.VMEM((CHUNK, 128), jnp.float32),
            pltpu.VMEM_SHARED((NROWS, 128), jnp.float32),
            pltpu.SemaphoreType.DMA,
            pltpu.SemaphoreType.DMA,
            pltpu.SemaphoreType.DMA,
            pltpu.SemaphoreType.DMA,
        ],
        interpret=interpret,
    )(_sc_edge_body)


_sc_deg = _make_sc_deg()
_sc_edge = _make_sc_edge()


def _mm(a, b):
    return jax.lax.dot_general(
        a, b, (((1,), (0,)), ((), ())),
        precision=jax.lax.Precision.HIGHEST,
        preferred_element_type=jnp.float32,
    )


def _dinv_of(deg2_ref):
    deg = deg2_ref[0, :, 0:1] + deg2_ref[1, :, 0:1] + 1.0
    return jax.lax.rsqrt(deg)


def _tc_h1_body(x_ref, w_ref, o_ref):
    o_ref[...] = _mm(x_ref[...], w_ref[...])


def _tc_h1(x, W1):
    return pl.pallas_call(
        _tc_h1_body,
        grid=(NBLK,),
        in_specs=[
            pl.BlockSpec((RBLK, D), lambda i: (i, 0)),
            pl.BlockSpec((D, H), lambda i: (0, 0)),
        ],
        out_specs=pl.BlockSpec((RBLK, H), lambda i: (i, 0)),
        out_shape=jax.ShapeDtypeStruct((NROWS, H), jnp.float32),
    )(x, W1)


def _tc_scale_body(h_ref, deg2_ref, o_ref):
    o_ref[...] = h_ref[...] * _dinv_of(deg2_ref)


def _tc_scale(h1, deg2):
    return pl.pallas_call(
        _tc_scale_body,
        grid=(NBLK,),
        in_specs=[
            pl.BlockSpec((RBLK, H), lambda i: (i, 0)),
            pl.BlockSpec((NC, RBLK, DEGW), lambda i: (0, i, 0)),
        ],
        out_specs=pl.BlockSpec((RBLK, H), lambda i: (i, 0)),
        out_shape=jax.ShapeDtypeStruct((NROWS, H), jnp.float32),
    )(h1, deg2)


def _tc_layer_body(e_ref, hs_ref, deg2_ref, w_ref, b_ref, o_ref):
    dinv = _dinv_of(deg2_ref)
    agg = e_ref[0] + e_ref[1] - hs_ref[...]
    z = agg * dinv + b_ref[...]
    r = jnp.maximum(z, 0.0)
    o_ref[...] = _mm(r, w_ref[...]) * dinv


def _tc_layer(e1, hs1, deg2, W2, b1):
    return pl.pallas_call(
        _tc_layer_body,
        grid=(NBLK,),
        in_specs=[
            pl.BlockSpec((NC, RBLK, H), lambda i: (0, i, 0)),
            pl.BlockSpec((RBLK, H), lambda i: (i, 0)),
            pl.BlockSpec((NC, RBLK, DEGW), lambda i: (0, i, 0)),
            pl.BlockSpec((H, H), lambda i: (0, 0)),
            pl.BlockSpec((1, H), lambda i: (0, 0)),
        ],
        out_specs=pl.BlockSpec((RBLK, H), lambda i: (i, 0)),
        out_shape=jax.ShapeDtypeStruct((NROWS, H), jnp.float32),
    )(e1, hs1, deg2, W2, b1)


def _tc_final_body(e_ref, hs_ref, deg2_ref, bat_ref, b_ref, fcw_ref, fcb_ref,
                   o_ref, sums, cnt):
    i = pl.program_id(0)
    dinv = _dinv_of(deg2_ref)
    agg = e_ref[0] + e_ref[1] - hs_ref[...]
    z = agg * dinv + b_ref[...]
    r = jnp.maximum(z, 0.0)

    bids = bat_ref[0, 0, :]
    seg = lax.broadcasted_iota(jnp.int32, (B, RBLK), 0)
    mask = (bids[None, :] == seg).astype(jnp.float32)

    @pl.when(i == 0)
    def _():
        sums[...] = jnp.zeros_like(sums)
        cnt[...] = jnp.zeros_like(cnt)

    sums[...] += _mm(mask, r)
    cnt[...] += jnp.broadcast_to(
        jnp.sum(mask, axis=1, keepdims=True), cnt.shape)

    @pl.when(i == NBLK - 1)
    def _():
        pooled = sums[...] / jnp.maximum(cnt[...], 1.0)
        o_ref[...] = _mm(pooled, fcw_ref[...]) + fcb_ref[...]


def _tc_final(e2, hs2, deg2, batch3, b2, fcW, fcb):
    return pl.pallas_call(
        _tc_final_body,
        grid=(NBLK,),
        in_specs=[
            pl.BlockSpec((NC, RBLK, H), lambda i: (0, i, 0)),
            pl.BlockSpec((RBLK, H), lambda i: (i, 0)),
            pl.BlockSpec((NC, RBLK, DEGW), lambda i: (0, i, 0)),
            pl.BlockSpec((1, 1, RBLK), lambda i: (i, 0, 0)),
            pl.BlockSpec((1, H), lambda i: (0, 0)),
            pl.BlockSpec((H, O), lambda i: (0, 0)),
            pl.BlockSpec((1, O), lambda i: (0, 0)),
        ],
        out_specs=pl.BlockSpec((B, O), lambda i: (0, 0)),
        out_shape=jax.ShapeDtypeStruct((B, O), jnp.float32),
        scratch_shapes=[
            pltpu.VMEM((B, H), jnp.float32),
            pltpu.VMEM((B, H), jnp.float32),
        ],
    )(e2, hs2, deg2, batch3, b2, fcW, fcb)


@jax.jit
def kernel(x, edge_index, batch, W1, b1, W2, b2, fcW, fcb):
    src = edge_index[0]
    dst = edge_index[1]
    pad_idx = N + jnp.arange(EPAD - E, dtype=jnp.int32) % (NROWS - N)
    srcp = jnp.concatenate([src, pad_idx]).reshape(NW, NG, G, CHUNK)
    dstp = jnp.concatenate([dst, pad_idx]).reshape(NW, NG, G, CHUNK)
    dstp3 = dstp.reshape(NW, CHUNKS, CHUNK)

    xp = jnp.concatenate([x, jnp.zeros((NROWS - N, D), jnp.float32)], axis=0)
    batp = jnp.concatenate(
        [batch, jnp.full((NROWS - N,), B, jnp.int32)]).reshape(NBLK, 1, RBLK)

    ones128 = jnp.ones((CHUNK, DEGW), jnp.float32)
    z16 = jnp.zeros((NROWS, DEGW), jnp.float32)

    deg2 = _sc_deg(dstp3, ones128, z16)
    h1 = _tc_h1(xp, W1)
    hs1 = _tc_scale(h1, deg2)
    e1 = _sc_edge(hs1, srcp, dstp)
    hs2 = _tc_layer(e1, hs1, deg2, W2, b1.reshape(1, H))
    e2 = _sc_edge(hs2, srcp, dstp)
    return _tc_final(e2, hs2, deg2, batp, b2.reshape(1, H),
                     fcW, fcb.reshape(1, O))
